# Initial kernel scaffold; baseline (speedup 1.0000x reference)
import jax
import jax.numpy as jnp
from jax.experimental import pallas as pl


def kernel(x, edge_index, W1, b1, W2, b2):
    raise NotImplementedError("write your pallas kernel here")


# speedup vs baseline: 14.2574x; 14.2574x over previous
import functools

import jax
import jax.numpy as jnp
from jax import lax
from jax.experimental import pallas as pl
from jax.experimental.pallas import tpu as pltpu
from jax.experimental.pallas import tpu_sc as plsc

N = 10000
E = 320000
D_IN = 128
D_H = 128
D_OUT = 64

NC = 2
NS = 16
NW = NC * NS
EPW = E // NW
K = 80
NPAD = 10112
RPT = NPAD // NS
ZR = RPT // 8

_mesh = lambda: plsc.VectorSubcoreMesh(core_axis_name="c", subcore_axis_name="s")


@functools.partial(
    pl.kernel,
    out_type=jax.ShapeDtypeStruct((NC, NS, NPAD), jnp.float32),
    mesh=_mesh(),
    compiler_params=pltpu.CompilerParams(needs_layout_passes=False),
    scratch_types=[
        pltpu.VMEM((EPW,), jnp.int32),
        pltpu.VMEM((NPAD,), jnp.float32),
    ],
)
def _sc_deg(dst_hbm, out_hbm, idx_v, cnt_v):
    c = lax.axis_index("c")
    s = lax.axis_index("s")
    base = (c * NS + s) * EPW
    pltpu.sync_copy(dst_hbm.at[pl.ds(base, EPW)], idx_v)

    def zero(i, carry):
        cnt_v[pl.ds(i * 16, 16)] = jnp.zeros((16,), jnp.float32)
        return carry

    lax.fori_loop(0, NPAD // 16, zero, 0)

    ones16 = jnp.ones((16,), jnp.float32)

    def hist(i, carry):
        idx = idx_v[pl.ds(i * 16, 16)]
        plsc.addupdate_scatter(cnt_v, [idx], ones16)
        return carry

    lax.fori_loop(0, EPW // 16, hist, 0)
    pltpu.sync_copy(cnt_v, out_hbm.at[c, s])


def _make_segsum(D):
    @functools.partial(
        pl.kernel,
        out_type=jax.ShapeDtypeStruct((NC, NPAD, D), jnp.float32),
        mesh=_mesh(),
        compiler_params=pltpu.CompilerParams(needs_layout_passes=False),
        scratch_types=[
            pltpu.VMEM((K,), jnp.int32),
            pltpu.VMEM((K,), jnp.int32),
            pltpu.VMEM((K, D), jnp.float32),
            pltpu.VMEM((ZR, D), jnp.float32),
            pltpu.VMEM_SHARED((NPAD, D), jnp.float32),
            pltpu.SemaphoreType.DMA,
        ],
    )
    def seg(src_hbm, dst_hbm, h_hbm, out_hbm, src_v, dst_v, rows_v, zero_v, acc_sh, sem):
        c = lax.axis_index("c")
        s = lax.axis_index("s")
        base = (c * NS + s) * EPW

        def zero(i, carry):
            r = i // (D // 16)
            l = i % (D // 16)
            zero_v[r, pl.ds(l * 16, 16)] = jnp.zeros((16,), jnp.float32)
            return carry

        lax.fori_loop(0, ZR * (D // 16), zero, 0)
        for t in range(RPT // ZR):
            pltpu.sync_copy(zero_v, acc_sh.at[pl.ds((s * (RPT // ZR) + t) * ZR, ZR)])
        plsc.subcore_barrier()

        def body(j, carry):
            off = base + j * K
            pltpu.sync_copy(src_hbm.at[pl.ds(off, K)], src_v)
            pltpu.sync_copy(dst_hbm.at[pl.ds(off, K)], dst_v)
            pltpu.async_copy(h_hbm.at[src_v], rows_v, sem).wait()
            pltpu.sync_copy(rows_v, acc_sh.at[dst_v], add=True)
            return carry

        lax.fori_loop(0, EPW // K, body, 0)
        plsc.subcore_barrier()
        pltpu.sync_copy(acc_sh.at[pl.ds(s * RPT, RPT)], out_hbm.at[c, pl.ds(s * RPT, RPT)])

    return seg


_sc_seg128 = _make_segsum(D_H)


def _tc_dis_body(parts_ref, o_ref):
    deg = jnp.sum(parts_ref[...], axis=0) + 1.0
    o_ref[...] = lax.rsqrt(deg)


def _tc_dis(parts):
    return pl.pallas_call(
        _tc_dis_body,
        out_shape=jax.ShapeDtypeStruct((NPAD,), jnp.float32),
    )(parts)


_R = 2000


def _tc_h1p_body(x_ref, w_ref, dis_ref, o_ref):
    h = jnp.dot(x_ref[...], w_ref[...], preferred_element_type=jnp.float32)
    o_ref[...] = h * dis_ref[...]


def _tc_h1p(x, W1, dis_col):
    return pl.pallas_call(
        _tc_h1p_body,
        grid=(N // _R,),
        in_specs=[
            pl.BlockSpec((_R, D_IN), lambda i: (i, 0)),
            pl.BlockSpec((D_IN, D_H), lambda i: (0, 0)),
            pl.BlockSpec((_R, 1), lambda i: (i, 0)),
        ],
        out_specs=pl.BlockSpec((_R, D_H), lambda i: (i, 0)),
        out_shape=jax.ShapeDtypeStruct((N, D_H), jnp.float32),
    )(x, W1, dis_col)


def _tc_u_body(agg_ref, hp_ref, dis_ref, b_ref, o_ref):
    d = dis_ref[...]
    z = d * (agg_ref[0] + agg_ref[1] + hp_ref[...]) + b_ref[...]
    o_ref[...] = jnp.maximum(z, 0.0) * d


def _tc_u(agg1, h1p, dis_col, b1_row):
    return pl.pallas_call(
        _tc_u_body,
        grid=(N // _R,),
        in_specs=[
            pl.BlockSpec((NC, _R, D_H), lambda i: (0, i, 0)),
            pl.BlockSpec((_R, D_H), lambda i: (i, 0)),
            pl.BlockSpec((_R, 1), lambda i: (i, 0)),
            pl.BlockSpec((1, D_H), lambda i: (0, 0)),
        ],
        out_specs=pl.BlockSpec((_R, D_H), lambda i: (i, 0)),
        out_shape=jax.ShapeDtypeStruct((N, D_H), jnp.float32),
    )(agg1, h1p, dis_col, b1_row)


def _tc_out_body(agg_ref, u_ref, dis_ref, w_ref, b_ref, o_ref):
    v = dis_ref[...] * (agg_ref[0] + agg_ref[1] + u_ref[...])
    z = jnp.dot(v, w_ref[...], preferred_element_type=jnp.float32) + b_ref[...]
    m = jnp.max(z, axis=1, keepdims=True)
    zm = z - m
    lse = jnp.log(jnp.sum(jnp.exp(zm), axis=1, keepdims=True))
    o_ref[...] = zm - lse


def _tc_out(agg2, u, dis_col, W2, b2_row):
    return pl.pallas_call(
        _tc_out_body,
        grid=(N // _R,),
        in_specs=[
            pl.BlockSpec((NC, _R, D_H), lambda i: (0, i, 0)),
            pl.BlockSpec((_R, D_H), lambda i: (i, 0)),
            pl.BlockSpec((_R, 1), lambda i: (i, 0)),
            pl.BlockSpec((D_H, D_OUT), lambda i: (0, 0)),
            pl.BlockSpec((1, D_OUT), lambda i: (0, 0)),
        ],
        out_specs=pl.BlockSpec((_R, D_OUT), lambda i: (i, 0)),
        out_shape=jax.ShapeDtypeStruct((N, D_OUT), jnp.float32),
    )(agg2, u, dis_col, W2, b2_row)


@jax.jit
def kernel(x, edge_index, W1, b1, W2, b2):
    src = edge_index[0]
    dst = edge_index[1]
    parts = _sc_deg(dst)
    dis = _tc_dis(parts.reshape(NW, NPAD))
    dis_col = dis.reshape(NPAD, 1)
    h1p = _tc_h1p(x, W1, dis_col)
    agg1 = _sc_seg128(src, dst, h1p)
    u = _tc_u(agg1, h1p, dis_col, b1.reshape(1, D_H))
    agg2 = _sc_seg128(src, dst, u)
    return _tc_out(agg2, u, dis_col, W2, b2.reshape(1, D_OUT))
